# baseline (speedup 1.0000x reference)
import jax
import jax.numpy as jnp
from jax.experimental import pallas as pl
from jax.experimental.pallas import tpu as pltpu

N = 10000
D = 128
ALPHA = 0.2
BM = 400
BH = BM // 2


def _pass1_body(al_ref, ar_ref, x_ref, w1_ref, w3_ref, b1_ref, s3_ref, s1_scr):
    @pl.when(pl.program_id(0) == 0)
    def _():
        s1_scr[...] = jnp.dot(
            x_ref[...], w1_ref[...], preferred_element_type=jnp.float32
        ).astype(jnp.bfloat16)

    for h, a_ref in enumerate((al_ref, ar_ref)):
        acc = jnp.dot(
            a_ref[...].astype(jnp.bfloat16), s1_scr[...],
            preferred_element_type=jnp.float32,
        )
        acc = acc + b1_ref[...]
        u1 = jnp.where(acc > 0, acc, acc * ALPHA)
        s3_ref[h * BH:(h + 1) * BH, :] = jnp.dot(
            u1.astype(jnp.bfloat16), w3_ref[...],
            preferred_element_type=jnp.float32,
        ).astype(jnp.bfloat16)


def _pass2_body(al_ref, ar_ref, x_ref, w2_ref, w4_ref, s3_ref, b2_ref, b3_ref,
                uo_ref, s4_ref, s23_scr):
    @pl.when(pl.program_id(0) == 0)
    def _():
        s23_scr[:, :D] = jnp.dot(
            x_ref[...], w2_ref[...], preferred_element_type=jnp.float32
        ).astype(jnp.bfloat16)
        s23_scr[:, D:] = s3_ref[...]

    for h, a_ref in enumerate((al_ref, ar_ref)):
        acc = jnp.dot(
            a_ref[...].astype(jnp.bfloat16), s23_scr[...],
            preferred_element_type=jnp.float32,
        )
        acc2 = acc[:, :D] + b2_ref[...]
        i1 = jnp.where(acc2 > 0, acc2, acc2 * ALPHA)
        s4_ref[h * BH:(h + 1) * BH, :] = jnp.dot(
            i1.astype(jnp.bfloat16), w4_ref[...],
            preferred_element_type=jnp.float32,
        ).astype(jnp.bfloat16)
        uo_ref[h * BH:(h + 1) * BH, :] = jnp.maximum(
            acc[:, D:] + b3_ref[...], 0.0)


def _pass3_body(al_ref, ar_ref, s4_ref, b4_ref, io_ref):
    for h, a_ref in enumerate((al_ref, ar_ref)):
        acc = jnp.dot(
            a_ref[...].astype(jnp.bfloat16), s4_ref[...],
            preferred_element_type=jnp.float32,
        )
        io_ref[h * BH:(h + 1) * BH, :] = jnp.maximum(acc + b4_ref[...], 0.0)


_FULL = pl.BlockSpec((N, D), lambda i: (0, 0))
_ROW = pl.BlockSpec((1, D), lambda i: (0, 0))
_W = pl.BlockSpec((D, D), lambda i: (0, 0))
_ADJ_L = pl.BlockSpec((BH, N), lambda i: (2 * i, 0))
_ADJ_R = pl.BlockSpec((BH, N), lambda i: (2 * i + 1, 0))
_OUT = pl.BlockSpec((BM, D), lambda i: (i, 0))


def kernel(ufea, vfea, UV_adj, VU_adj, W1, W2, W3, W4, b1, b2, b3, b4):
    w3b = W3.astype(jnp.bfloat16)
    w4b = W4.astype(jnp.bfloat16)
    grid = (N // BM,)

    s3 = pl.pallas_call(
        _pass1_body,
        grid=grid,
        in_specs=[_ADJ_L, _ADJ_R, _FULL, _W, _W, _ROW],
        out_specs=_OUT,
        out_shape=jax.ShapeDtypeStruct((N, D), jnp.bfloat16),
        scratch_shapes=[pltpu.VMEM((N, D), jnp.bfloat16)],
    )(VU_adj, VU_adj, ufea, W1, w3b, b1.reshape(1, D))

    U_out, s4 = pl.pallas_call(
        _pass2_body,
        grid=grid,
        in_specs=[_ADJ_L, _ADJ_R, _FULL, _W, _W, _FULL, _ROW, _ROW],
        out_specs=[_OUT, _OUT],
        out_shape=[
            jax.ShapeDtypeStruct((N, D), jnp.float32),
            jax.ShapeDtypeStruct((N, D), jnp.bfloat16),
        ],
        scratch_shapes=[pltpu.VMEM((N, 2 * D), jnp.bfloat16)],
    )(UV_adj, UV_adj, vfea, W2, w4b, s3, b2.reshape(1, D), b3.reshape(1, D))

    I_out = pl.pallas_call(
        _pass3_body,
        grid=grid,
        in_specs=[_ADJ_L, _ADJ_R, _FULL, _ROW],
        out_specs=_OUT,
        out_shape=jax.ShapeDtypeStruct((N, D), jnp.float32),
    )(VU_adj, VU_adj, s4, b4.reshape(1, D))

    return U_out, I_out


# speedup vs baseline: 1.0218x; 1.0218x over previous
import jax
import jax.numpy as jnp
from jax.experimental import pallas as pl
from jax.experimental.pallas import tpu as pltpu

N = 10000
D = 128
ALPHA = 0.2
BM = 200
NB = N // BM


def _body(vu_ref, uv_ref, u_ref, v_ref, w1_ref, w2_ref, w3_ref, w4_ref,
          b_ref, uo_ref, io_ref, s1_scr, s23_scr, s4_scr):
    i = pl.program_id(0)
    phase = i // NB
    j = i % NB

    @pl.when(i == 0)
    def _():
        s1_scr[...] = jnp.dot(
            u_ref[...], w1_ref[...], preferred_element_type=jnp.float32
        ).astype(jnp.bfloat16)
        s23_scr[:, :D] = jnp.dot(
            v_ref[...], w2_ref[...], preferred_element_type=jnp.float32
        ).astype(jnp.bfloat16)

    @pl.when(phase == 0)
    def _():
        a = vu_ref[...].astype(jnp.bfloat16)
        acc = jnp.dot(a, s1_scr[...], preferred_element_type=jnp.float32)
        acc = acc + b_ref[0, :]
        u1 = jnp.where(acc > 0, acc, acc * ALPHA)
        s23_scr[pl.ds(j * BM, BM), D:] = jnp.dot(
            u1.astype(jnp.bfloat16), w3_ref[...],
            preferred_element_type=jnp.float32,
        ).astype(jnp.bfloat16)

    @pl.when(phase == 1)
    def _():
        a = uv_ref[...].astype(jnp.bfloat16)
        acc = jnp.dot(a, s23_scr[...], preferred_element_type=jnp.float32)
        acc2 = acc[:, :D] + b_ref[1, :]
        i1 = jnp.where(acc2 > 0, acc2, acc2 * ALPHA)
        s4_scr[pl.ds(j * BM, BM), :] = jnp.dot(
            i1.astype(jnp.bfloat16), w4_ref[...],
            preferred_element_type=jnp.float32,
        ).astype(jnp.bfloat16)
        uo_ref[...] = jnp.maximum(acc[:, D:] + b_ref[2, :], 0.0)

    @pl.when(phase == 2)
    def _():
        a = vu_ref[...].astype(jnp.bfloat16)
        acc = jnp.dot(a, s4_scr[...], preferred_element_type=jnp.float32)
        io_ref[...] = jnp.maximum(acc + b_ref[3, :], 0.0)


def _vu_map(i):
    return (jnp.where(i < NB, i, jnp.where(i < 2 * NB, NB - 1, i - 2 * NB)), 0)


def _uv_map(i):
    return (jnp.clip(i - NB, 0, NB - 1), 0)


def _uo_map(i):
    return (jnp.clip(i - NB, 0, NB - 1), 0)


def _io_map(i):
    return (jnp.clip(i - 2 * NB, 0, NB - 1), 0)


_FULL = pl.BlockSpec((N, D), lambda i: (0, 0))
_W = pl.BlockSpec((D, D), lambda i: (0, 0))


def kernel(ufea, vfea, UV_adj, VU_adj, W1, W2, W3, W4, b1, b2, b3, b4):
    w3b = W3.astype(jnp.bfloat16)
    w4b = W4.astype(jnp.bfloat16)
    b = jnp.stack([b1, b2, b3, b4])

    U_out, I_out = pl.pallas_call(
        _body,
        grid=(3 * NB,),
        in_specs=[
            pl.BlockSpec((BM, N), _vu_map),
            pl.BlockSpec((BM, N), _uv_map),
            _FULL, _FULL, _W, _W, _W, _W,
            pl.BlockSpec((4, D), lambda i: (0, 0)),
        ],
        out_specs=[
            pl.BlockSpec((BM, D), _uo_map),
            pl.BlockSpec((BM, D), _io_map),
        ],
        out_shape=[
            jax.ShapeDtypeStruct((N, D), jnp.float32),
            jax.ShapeDtypeStruct((N, D), jnp.float32),
        ],
        scratch_shapes=[
            pltpu.VMEM((N, D), jnp.bfloat16),
            pltpu.VMEM((N, 2 * D), jnp.bfloat16),
            pltpu.VMEM((N, D), jnp.bfloat16),
        ],
    )(VU_adj, UV_adj, ufea, vfea, W1, W2, w3b, w4b, b)

    return U_out, I_out


# speedup vs baseline: 1.0701x; 1.0473x over previous
import jax
import jax.numpy as jnp
from jax.experimental import pallas as pl
from jax.experimental.pallas import tpu as pltpu

N = 10000
D = 128
ALPHA = 0.2
BM = 400
NB = N // BM
NSTREAM = 2
BS = BM // NSTREAM


def _body(vu_ref, uv_ref, u_ref, v_ref, w1_ref, w2_ref, w3_ref, w4_ref,
          b_ref, uo_ref, io_ref, abuf, s1_scr, s23_scr, s4_scr, sem):
    i = pl.program_id(0)
    phase = i // NB
    j = i % NB

    def start_fetch(step):
        ph = step // NB
        row = (step % NB) * BM
        slot = step % 2

        @pl.when(ph != 1)
        def _():
            for h in range(NSTREAM):
                pltpu.make_async_copy(
                    vu_ref.at[pl.ds(row + h * BS, BS), :],
                    abuf.at[slot, pl.ds(h * BS, BS), :],
                    sem.at[slot, h],
                ).start()

        @pl.when(ph == 1)
        def _():
            for h in range(NSTREAM):
                pltpu.make_async_copy(
                    uv_ref.at[pl.ds(row + h * BS, BS), :],
                    abuf.at[slot, pl.ds(h * BS, BS), :],
                    sem.at[slot, h],
                ).start()

    @pl.when(i == 0)
    def _():
        start_fetch(i)
        s1_scr[...] = jnp.dot(
            u_ref[...], w1_ref[...], preferred_element_type=jnp.float32
        ).astype(jnp.bfloat16)
        s23_scr[:, :D] = jnp.dot(
            v_ref[...], w2_ref[...], preferred_element_type=jnp.float32
        ).astype(jnp.bfloat16)

    @pl.when(i + 1 < 3 * NB)
    def _():
        start_fetch(i + 1)

    for h in range(NSTREAM):
        pltpu.make_async_copy(
            vu_ref.at[pl.ds(h * BS, BS), :],
            abuf.at[i % 2, pl.ds(h * BS, BS), :],
            sem.at[i % 2, h],
        ).wait()

    @pl.when(phase == 0)
    def _():
        a = abuf[i % 2].astype(jnp.bfloat16)
        acc = jnp.dot(a, s1_scr[...], preferred_element_type=jnp.float32)
        acc = acc + b_ref[0, :]
        u1 = jnp.where(acc > 0, acc, acc * ALPHA)
        s23_scr[pl.ds(j * BM, BM), D:] = jnp.dot(
            u1.astype(jnp.bfloat16), w3_ref[...],
            preferred_element_type=jnp.float32,
        ).astype(jnp.bfloat16)

    @pl.when(phase == 1)
    def _():
        a = abuf[i % 2].astype(jnp.bfloat16)
        acc = jnp.dot(a, s23_scr[...], preferred_element_type=jnp.float32)
        acc2 = acc[:, :D] + b_ref[1, :]
        i1 = jnp.where(acc2 > 0, acc2, acc2 * ALPHA)
        s4_scr[pl.ds(j * BM, BM), :] = jnp.dot(
            i1.astype(jnp.bfloat16), w4_ref[...],
            preferred_element_type=jnp.float32,
        ).astype(jnp.bfloat16)
        uo_ref[...] = jnp.maximum(acc[:, D:] + b_ref[2, :], 0.0)

    @pl.when(phase == 2)
    def _():
        a = abuf[i % 2].astype(jnp.bfloat16)
        acc = jnp.dot(a, s4_scr[...], preferred_element_type=jnp.float32)
        io_ref[...] = jnp.maximum(acc + b_ref[3, :], 0.0)


def _uo_map(i):
    return (jnp.clip(i - NB, 0, NB - 1), 0)


def _io_map(i):
    return (jnp.clip(i - 2 * NB, 0, NB - 1), 0)


_FULL = pl.BlockSpec((N, D), lambda i: (0, 0))
_W = pl.BlockSpec((D, D), lambda i: (0, 0))


def kernel(ufea, vfea, UV_adj, VU_adj, W1, W2, W3, W4, b1, b2, b3, b4):
    w3b = W3.astype(jnp.bfloat16)
    w4b = W4.astype(jnp.bfloat16)
    b = jnp.stack([b1, b2, b3, b4])

    U_out, I_out = pl.pallas_call(
        _body,
        grid=(3 * NB,),
        in_specs=[
            pl.BlockSpec(memory_space=pl.ANY),
            pl.BlockSpec(memory_space=pl.ANY),
            _FULL, _FULL, _W, _W, _W, _W,
            pl.BlockSpec((4, D), lambda i: (0, 0)),
        ],
        out_specs=[
            pl.BlockSpec((BM, D), _uo_map),
            pl.BlockSpec((BM, D), _io_map),
        ],
        out_shape=[
            jax.ShapeDtypeStruct((N, D), jnp.float32),
            jax.ShapeDtypeStruct((N, D), jnp.float32),
        ],
        scratch_shapes=[
            pltpu.VMEM((2, BM, N), jnp.float32),
            pltpu.VMEM((N, D), jnp.bfloat16),
            pltpu.VMEM((N, 2 * D), jnp.bfloat16),
            pltpu.VMEM((N, D), jnp.bfloat16),
            pltpu.SemaphoreType.DMA((2, NSTREAM)),
        ],
    )(VU_adj, UV_adj, ufea, vfea, W1, W2, w3b, w4b, b)

    return U_out, I_out
